# Initial kernel scaffold; baseline (speedup 1.0000x reference)
import jax
import jax.numpy as jnp
from jax.experimental import pallas as pl


def kernel(inputs, unpool_idx):
    raise NotImplementedError("write your pallas kernel here")


# speedup vs baseline: 2.9355x; 2.9355x over previous
import functools

import jax
import jax.numpy as jnp
from jax import lax
from jax.experimental import pallas as pl
from jax.experimental.pallas import tpu as pltpu
from jax.experimental.pallas import tpu_sc as plsc

_B, _N, _D = 2, 10000, 128
_E = 160000
_NC, _NS, _L = 2, 16, 16
_NW = _NC * _NS
_EPW = _E // _NW
_CHUNK = 40
_IDXC = 2 * _CHUNK
_NCHUNK = _EPW // _CHUNK
_CPROWS = 80
_NCPB = _N // _CPROWS


def _unpool_body(inp_hbm, idx_hbm, out_hbm, idx_v, rows_v, out_v, cp_v, sem):
    wid = lax.axis_index("s") * _NC + lax.axis_index("c")

    pltpu.sync_copy(idx_hbm.at[wid], idx_v)

    for b in range(_B):
        for k in range(-(-_NCPB // _NW)):
            cid = wid + k * _NW

            @pl.when(cid < _NCPB)
            def _copy():
                pltpu.sync_copy(inp_hbm.at[b].at[pl.ds(cid * _CPROWS, _CPROWS)],
                                cp_v)
                pltpu.sync_copy(cp_v,
                                out_hbm.at[b].at[pl.ds(cid * _CPROWS, _CPROWS)])

    for b in range(_B):
        table = inp_hbm.at[b]
        obase = _N + wid * _EPW

        @pl.loop(0, _NCHUNK)
        def _chunk(c):
            pltpu.async_copy(table.at[idx_v.at[c]], rows_v, sem).wait()

            @pl.loop(0, _CHUNK)
            def _edge(t):
                for d in range(_D // _L):
                    sl = pl.ds(d * _L, _L)
                    out_v[t, sl] = 0.5 * (rows_v[2 * t, sl] +
                                          rows_v[2 * t + 1, sl])

            pltpu.sync_copy(out_v,
                            out_hbm.at[b].at[pl.ds(obase + c * _CHUNK, _CHUNK)])


@jax.jit
def kernel(inputs, unpool_idx):
    idx3 = unpool_idx.reshape(_NW, _NCHUNK, _IDXC)
    mesh = plsc.VectorSubcoreMesh(core_axis_name="c", subcore_axis_name="s")
    run = pl.kernel(
        _unpool_body,
        out_type=jax.ShapeDtypeStruct((_B, _N + _E, _D), jnp.float32),
        mesh=mesh,
        scratch_types=[
            pltpu.VMEM((_NCHUNK, _IDXC), jnp.int32),
            pltpu.VMEM((_IDXC, _D), jnp.float32),
            pltpu.VMEM((_CHUNK, _D), jnp.float32),
            pltpu.VMEM((_CPROWS, _D), jnp.float32),
            pltpu.SemaphoreType.DMA,
        ],
    )
    return run(inputs, idx3)


# speedup vs baseline: 4.6939x; 1.5990x over previous
import functools

import jax
import jax.numpy as jnp
from jax import lax
from jax.experimental import pallas as pl
from jax.experimental.pallas import tpu as pltpu
from jax.experimental.pallas import tpu_sc as plsc

_B, _N, _D = 2, 10000, 128
_E = 160000
_NC, _NS, _L = 2, 16, 16
_NW = _NC * _NS
_EPW = _E // _NW
_CHUNK = 40
_IDXC = 2 * _CHUNK
_NCHUNK = _EPW // _CHUNK
_TCHUNK = _B * _NCHUNK
_CPROWS = 80
_NCPB = _N // _CPROWS
_CPK = -(-_NCPB // _NW)


def _unpool_body(inp_hbm, idx_hbm, out_hbm, idx_v, rows_v, out_v, gsem, ssem,
                 csem):
    wid = lax.axis_index("s") * _NC + lax.axis_index("c")

    def _cp_refs(b, k):
        cid = wid + k * _NW
        src = inp_hbm.at[pl.ds(b * _N + cid * _CPROWS, _CPROWS)]
        dst = out_hbm.at[pl.ds(b * (_N + _E) + cid * _CPROWS, _CPROWS)]
        return cid, src, dst

    for b in range(_B):
        for k in range(_CPK):
            cid, src, dst = _cp_refs(b, k)

            @pl.when(cid < _NCPB)
            def _fire():
                pltpu.async_copy(src, dst, csem)

    pltpu.sync_copy(idx_hbm.at[wid], idx_v)

    def _issue_gather(c, u):
        pltpu.async_copy(inp_hbm.at[idx_v.at[c]], rows_v.at[u], gsem.at[u])

    def _wait_gather(u):
        pltpu.make_async_copy(inp_hbm.at[pl.ds(0, _IDXC)], rows_v.at[u],
                              gsem.at[u]).wait()

    def _wait_store(u):
        pltpu.make_async_copy(inp_hbm.at[pl.ds(0, _CHUNK)], out_v.at[u],
                              ssem.at[u]).wait()

    _issue_gather(0, 0)
    _issue_gather(1, 1)

    @pl.loop(0, _TCHUNK, step=2)
    def _outer(c0):
        for u in range(2):
            c = c0 + u
            b = c & 1
            r = lax.shift_right_logical(c, 1)
            _wait_gather(u)

            @pl.when(c0 >= 2)
            def _drain_prev_store():
                _wait_store(u)

            @pl.loop(0, _CHUNK)
            def _edge(t):
                for d in range(_D // _L):
                    sl = pl.ds(d * _L, _L)
                    out_v[u, t, sl] = 0.5 * (rows_v[u, 2 * t, sl] +
                                             rows_v[u, 2 * t + 1, sl])

            @pl.when(c + 2 < _TCHUNK)
            def _next_gather():
                _issue_gather(c + 2, u)

            orow = b * (_N + _E) + _N + wid * _EPW + r * _CHUNK
            pltpu.async_copy(out_v.at[u], out_hbm.at[pl.ds(orow, _CHUNK)],
                             ssem.at[u])

    _wait_store(0)
    _wait_store(1)
    for b in range(_B):
        for k in range(_CPK):
            cid, src, dst = _cp_refs(b, k)

            @pl.when(cid < _NCPB)
            def _drain():
                pltpu.make_async_copy(src, dst, csem).wait()


@jax.jit
def kernel(inputs, unpool_idx):
    idx3 = unpool_idx.reshape(_NW, _NCHUNK, _IDXC)
    idx_all = jnp.stack([idx3, idx3 + _N], axis=2).reshape(_NW, _TCHUNK, _IDXC)
    mesh = plsc.VectorSubcoreMesh(core_axis_name="c", subcore_axis_name="s")
    run = pl.kernel(
        _unpool_body,
        out_type=jax.ShapeDtypeStruct((_B * (_N + _E), _D), jnp.float32),
        mesh=mesh,
        scratch_types=[
            pltpu.VMEM((_TCHUNK, _IDXC), jnp.int32),
            pltpu.VMEM((2, _IDXC, _D), jnp.float32),
            pltpu.VMEM((2, _CHUNK, _D), jnp.float32),
            pltpu.SemaphoreType.DMA((2,)),
            pltpu.SemaphoreType.DMA((2,)),
            pltpu.SemaphoreType.DMA,
        ],
    )
    out = run(inputs.reshape(_B * _N, _D), idx_all)
    return out.reshape(_B, _N + _E, _D)
